# baseline (speedup 1.0000x reference)
import functools

import jax
import jax.numpy as jnp
from jax import lax
from jax.experimental import pallas as pl
from jax.experimental.pallas import tpu as pltpu
from jax.experimental.pallas import tpu_sc as plsc

B = 4
V = 50000
F = 100000
NC = 2
NS = 16
L = 16
FPT = 6400
F_PAD = NS * FPT
STEP = 64
GRP = STEP // L
NSTEPS = FPT // STEP
APT = 8 * V // NS
VCH = 6256
VTAIL = 2 * V - (NS - 1) * VCH
XCH = 3200


def _iota():
    return lax.iota(jnp.int32, L)


def _rsqrt(x):
    i = lax.bitcast_convert_type(x, jnp.int32)
    i = jnp.int32(0x5F3759DF) - lax.shift_right_logical(
        i, jnp.full((L,), 1, jnp.int32))
    r = lax.bitcast_convert_type(i, jnp.float32)
    xh = 0.5 * x
    for _ in range(3):
        r = r * (1.5 - xh * r * r)
    return r


def _sqrt_pos(x):
    r = _rsqrt(jnp.maximum(x, 1e-30))
    return jnp.where(x > 0, x * r, 0.0)


def _lap_body(xt, ft, zz, out, fidx, gidx, sidx, rows, vals, svmem, abuf,
              accum, sg0, sg1, ss0, ss1):
    sg = (sg0, sg1)
    ss = (ss0, ss1)
    c = lax.axis_index("c")
    s = lax.axis_index("s")
    iota = _iota()

    pltpu.sync_copy(zz, abuf.at[pl.ds(0, APT)])
    pltpu.sync_copy(abuf.at[pl.ds(0, APT)], accum.at[pl.ds(s * APT, APT)])

    for j in range(3):
        pltpu.sync_copy(ft.at[pl.ds(j * F_PAD + s * FPT, FPT)],
                        fidx.at[pl.ds(j * FPT, FPT)])

    plsc.subcore_barrier()

    goff0 = c * 6 * V

    def build_gidx(p, t, soff, goff0=goff0):
        base = t * STEP

        def idx_body(g, _):
            o = g * L
            for j in range(3):
                fv = fidx[pl.ds(j * FPT + base + o, L)] + (goff0 + soff)
                for m in range(3):
                    gidx[p][j][m][pl.ds(o, L)] = fv + m * (2 * V)
            return 0

        lax.fori_loop(0, GRP, idx_body, 0)

    def fire_g(p):
        for j in range(3):
            for m in range(3):
                pltpu.async_copy(
                    xt.at[gidx[p][j][m]],
                    rows[p].at[pl.ds((j * 3 + m) * STEP, STEP)], sg[p])

    def wait_g(p):
        for j in range(3):
            for m in range(3):
                pltpu.make_async_copy(
                    xt.at[gidx[p][j][m]],
                    rows[p].at[pl.ds((j * 3 + m) * STEP, STEP)],
                    sg[p]).wait()

    def fire_s(p):
        for j in range(3):
            for m in range(4):
                pltpu.async_copy(
                    vals[p].at[pl.ds((j * 4 + m) * STEP, STEP)],
                    accum.at[sidx[p][j][m]], ss[p], add=True)

    def wait_s(p):
        for j in range(3):
            for m in range(4):
                pltpu.make_async_copy(
                    vals[p].at[pl.ds((j * 4 + m) * STEP, STEP)],
                    accum.at[sidx[p][j][m]], ss[p]).wait()

    def do_math(p, t, soff):
        base = t * STEP
        rw = rows[p]
        vw = vals[p]

        def math_body(g, _):
            o = g * L
            fvs = [fidx[pl.ds(j * FPT + base + o, L)] + soff for j in range(3)]
            for j in range(3):
                for m in range(4):
                    sidx[p][j][m][pl.ds(o, L)] = fvs[j] + m * (2 * V)

            def ld(j, m):
                return rw[pl.ds((j * 3 + m) * STEP + o, L)]

            x0 = [ld(0, m) for m in range(3)]
            x1 = [ld(1, m) for m in range(3)]
            x2 = [ld(2, m) for m in range(3)]
            eA = [x1[k] - x2[k] for k in range(3)]
            eB = [x0[k] - x2[k] for k in range(3)]
            eC = [x0[k] - x1[k] for k in range(3)]
            A2 = eA[0] * eA[0] + eA[1] * eA[1] + eA[2] * eA[2]
            B2 = eB[0] * eB[0] + eB[1] * eB[1] + eB[2] * eB[2]
            C2 = eC[0] * eC[0] + eC[1] * eC[1] + eC[2] * eC[2]
            An = _sqrt_pos(A2)
            Bn = _sqrt_pos(B2)
            Cn = _sqrt_pos(C2)
            sh = 0.5 * (An + Bn + Cn)
            prod = sh * (sh - An) * (sh - Bn) * (sh - Cn)
            prod = jnp.maximum(prod, 1e-12)
            rinv = _rsqrt(prod)
            area = prod * rinv
            cota = (B2 + C2 - A2) * rinv
            cotb = (A2 + C2 - B2) * rinv
            cotc = (A2 + B2 - C2) * rinv
            gid = s * FPT + base + o + iota
            m = gid < F
            area_m = jnp.where(m, area, 0.0)
            for k in range(3):
                d1 = cotc * (x0[k] - x1[k]) + cota * (x2[k] - x1[k])
                d2 = cota * (x1[k] - x2[k]) + cotb * (x0[k] - x2[k])
                d0 = -(d1 + d2)
                vw[pl.ds(0 * STEP * 4 + k * STEP + o, L)] = jnp.where(m, d0, 0.0)
                vw[pl.ds(1 * STEP * 4 + k * STEP + o, L)] = jnp.where(m, d1, 0.0)
                vw[pl.ds(2 * STEP * 4 + k * STEP + o, L)] = jnp.where(m, d2, 0.0)
            for j in range(3):
                vw[pl.ds(j * STEP * 4 + 3 * STEP + o, L)] = area_m
            return 0

        lax.fori_loop(0, GRP, math_body, 0)

    for b01 in range(2):
        soff = b01 * V

        build_gidx(0, 0, soff)
        fire_g(0)

        def pair_body(i, _, soff=soff):
            for sub in range(2):
                t = 2 * i + sub
                p = sub
                q = 1 - sub
                if sub == 0:
                    build_gidx(q, t + 1, soff)
                    fire_g(q)
                else:
                    @pl.when(i < NSTEPS // 2 - 1)
                    def _():
                        build_gidx(q, t + 1, soff)
                        fire_g(q)

                @pl.when(i > 0)
                def _():
                    wait_s(p)

                wait_g(p)
                do_math(p, t, soff)
                fire_s(p)
            return 0

        lax.fori_loop(0, NSTEPS // 2, pair_body, 0)
        wait_s(0)
        wait_s(1)

    plsc.subcore_barrier()

    for m in range(4):

        @pl.when(s < NS - 1)
        def _():
            pltpu.sync_copy(accum.at[pl.ds(m * 2 * V + s * VCH, VCH)],
                            abuf.at[pl.ds(m * VCH, VCH)])

        @pl.when(s == NS - 1)
        def _():
            pltpu.sync_copy(
                accum.at[pl.ds(m * 2 * V + (NS - 1) * VCH, VTAIL)],
                abuf.at[pl.ds(m * VCH, VTAIL)])

    nvalid = jnp.where(s == NS - 1, VTAIL, VCH)
    ngroups = VCH // L

    def vtx_body(g, acc):
        o = g * L
        rv = o + iota
        valid = rv < nvalid
        ax = abuf[pl.ds(0 * VCH + o, L)]
        ay = abuf[pl.ds(1 * VCH + o, L)]
        az = abuf[pl.ds(2 * VCH + o, L)]
        aa = abuf[pl.ds(3 * VCH + o, L)]
        w = jnp.where(aa > 0, 0.0625 / aa, 0.0)
        lx = ax * w
        ly = ay * w
        lz = az * w
        sq = lx * lx + ly * ly + lz * lz
        cv = _sqrt_pos(sq)
        cv = jnp.where(valid, cv, 0.0)
        return acc + cv

    csum = lax.fori_loop(0, ngroups, vtx_body, jnp.zeros((L,), jnp.float32))
    svmem[pl.ds(0, L)] = csum
    pltpu.sync_copy(svmem, out.at[pl.ds((c * NS + s) * L, L)])


_lap_kernel = functools.partial(
    pl.kernel,
    out_type=jax.ShapeDtypeStruct((NC * NS * L,), jnp.float32),
    mesh=plsc.VectorSubcoreMesh(
        core_axis_name="c", subcore_axis_name="s",
        num_cores=NC, num_subcores=NS),
    scratch_types=[
        pltpu.VMEM((3 * FPT,), jnp.int32),
        [[[pltpu.VMEM((STEP,), jnp.int32) for _ in range(3)]
          for _ in range(3)] for _ in range(2)],
        [[[pltpu.VMEM((STEP,), jnp.int32) for _ in range(4)]
          for _ in range(3)] for _ in range(2)],
        [pltpu.VMEM((9 * STEP,), jnp.float32)
         for _ in range(2)],
        [pltpu.VMEM((12 * STEP,), jnp.float32)
         for _ in range(2)],
        pltpu.VMEM((L,), jnp.float32),
        pltpu.VMEM((4 * VCH,), jnp.float32),
        pltpu.VMEM_SHARED((8 * V,), jnp.float32),
        pltpu.SemaphoreType.DMA,
        pltpu.SemaphoreType.DMA,
        pltpu.SemaphoreType.DMA,
        pltpu.SemaphoreType.DMA,
    ],
)(_lap_body)


def kernel(x, y, faces):
    del y
    xt = jnp.transpose(x.reshape(2, 2, V, 3), (0, 3, 1, 2)).reshape(-1)
    ft = jnp.pad(faces.T, ((0, 0), (0, F_PAD - F))).reshape(-1)
    zz = jnp.zeros((APT,), jnp.float32)
    parts = _lap_kernel(xt, ft, zz)
    return jnp.sum(parts) / (B * V)


# speedup vs baseline: 1.6361x; 1.6361x over previous
import functools

import jax
import jax.numpy as jnp
from jax import lax
from jax.experimental import pallas as pl
from jax.experimental.pallas import tpu as pltpu
from jax.experimental.pallas import tpu_sc as plsc

B = 4
V = 50000
F = 100000
NC = 2
NS = 16
L = 16
FPT = 6272
F_PAD = NS * FPT
STEP = 64
GRP = STEP // L
NSTEPS = FPT // STEP
APT = 8 * V // NS
VCH = 6256
VTAIL = 2 * V - (NS - 1) * VCH
XCH = 3200


def _iota():
    return lax.iota(jnp.int32, L)


def _rsqrt(x):
    i = lax.bitcast_convert_type(x, jnp.int32)
    i = jnp.int32(0x5F3759DF) - lax.shift_right_logical(
        i, jnp.full((L,), 1, jnp.int32))
    r = lax.bitcast_convert_type(i, jnp.float32)
    xh = 0.5 * x
    for _ in range(3):
        r = r * (1.5 - xh * r * r)
    return r


def _sqrt_pos(x):
    r = _rsqrt(jnp.maximum(x, 1e-30))
    return jnp.where(x > 0, x * r, 0.0)


def _lap_body(xt, ft, zz, out, fidx, gidx, sidx, rows, vals, svmem, abuf,
              xsrc, accum, sg0, sg1, ss0, ss1):
    sg = (sg0, sg1)
    ss = (ss0, ss1)
    c = lax.axis_index("c")
    s = lax.axis_index("s")
    iota = _iota()

    pltpu.sync_copy(zz, abuf.at[pl.ds(0, APT)])
    pltpu.sync_copy(abuf.at[pl.ds(0, APT)], accum.at[pl.ds(s * APT, APT)])

    for b01 in range(2):
        for m in range(3):
            src_base = (2 * c + b01) * 3 * V + m * V
            dst_base = m * 2 * V + b01 * V

            @pl.when(s < NS - 1)
            def _():
                pltpu.sync_copy(xt.at[pl.ds(src_base + s * XCH, XCH)],
                                abuf.at[pl.ds(0, XCH)])
                pltpu.sync_copy(abuf.at[pl.ds(0, XCH)],
                                xsrc.at[pl.ds(dst_base + s * XCH, XCH)])

            @pl.when(s == NS - 1)
            def _():
                tail = V - (NS - 1) * XCH
                pltpu.sync_copy(xt.at[pl.ds(src_base + (NS - 1) * XCH, tail)],
                                abuf.at[pl.ds(0, tail)])
                pltpu.sync_copy(abuf.at[pl.ds(0, tail)],
                                xsrc.at[pl.ds(dst_base + (NS - 1) * XCH, tail)])

    for j in range(3):
        pltpu.sync_copy(ft.at[pl.ds(j * F_PAD + s * FPT, FPT)],
                        fidx.at[pl.ds(j * FPT, FPT)])

    plsc.subcore_barrier()

    def build_gidx(p, t, soff):
        base = t * STEP

        def idx_body(g, _):
            o = g * L
            for j in range(3):
                fv = fidx[pl.ds(j * FPT + base + o, L)] + soff
                for m in range(3):
                    gidx[p][j][m][pl.ds(o, L)] = fv + m * (2 * V)
            return 0

        lax.fori_loop(0, GRP, idx_body, 0)

    def fire_g(p):
        for j in range(3):
            for m in range(3):
                pltpu.async_copy(
                    xsrc.at[gidx[p][j][m]],
                    rows[p].at[pl.ds((j * 3 + m) * STEP, STEP)], sg[p])

    def wait_g(p):
        for j in range(3):
            for m in range(3):
                pltpu.make_async_copy(
                    xsrc.at[gidx[p][j][m]],
                    rows[p].at[pl.ds((j * 3 + m) * STEP, STEP)],
                    sg[p]).wait()

    def fire_s(p):
        for j in range(3):
            for m in range(4):
                pltpu.async_copy(
                    vals[p].at[pl.ds((j * 4 + m) * STEP, STEP)],
                    accum.at[sidx[p][j][m]], ss[p], add=True)

    def wait_s(p):
        for j in range(3):
            for m in range(4):
                pltpu.make_async_copy(
                    vals[p].at[pl.ds((j * 4 + m) * STEP, STEP)],
                    accum.at[sidx[p][j][m]], ss[p]).wait()

    def do_math(p, t, soff):
        base = t * STEP
        rw = rows[p]
        vw = vals[p]

        def math_body(g, _):
            o = g * L
            fvs = [fidx[pl.ds(j * FPT + base + o, L)] + soff for j in range(3)]
            for j in range(3):
                for m in range(4):
                    sidx[p][j][m][pl.ds(o, L)] = fvs[j] + m * (2 * V)

            def ld(j, m):
                return rw[pl.ds((j * 3 + m) * STEP + o, L)]

            x0 = [ld(0, m) for m in range(3)]
            x1 = [ld(1, m) for m in range(3)]
            x2 = [ld(2, m) for m in range(3)]
            eA = [x1[k] - x2[k] for k in range(3)]
            eB = [x0[k] - x2[k] for k in range(3)]
            eC = [x0[k] - x1[k] for k in range(3)]
            A2 = eA[0] * eA[0] + eA[1] * eA[1] + eA[2] * eA[2]
            B2 = eB[0] * eB[0] + eB[1] * eB[1] + eB[2] * eB[2]
            C2 = eC[0] * eC[0] + eC[1] * eC[1] + eC[2] * eC[2]
            An = _sqrt_pos(A2)
            Bn = _sqrt_pos(B2)
            Cn = _sqrt_pos(C2)
            sh = 0.5 * (An + Bn + Cn)
            prod = sh * (sh - An) * (sh - Bn) * (sh - Cn)
            prod = jnp.maximum(prod, 1e-12)
            rinv = _rsqrt(prod)
            area = prod * rinv
            cota = (B2 + C2 - A2) * rinv
            cotb = (A2 + C2 - B2) * rinv
            cotc = (A2 + B2 - C2) * rinv
            gid = s * FPT + base + o + iota
            m = gid < F
            area_m = jnp.where(m, area, 0.0)
            for k in range(3):
                d1 = cotc * (x0[k] - x1[k]) + cota * (x2[k] - x1[k])
                d2 = cota * (x1[k] - x2[k]) + cotb * (x0[k] - x2[k])
                d0 = -(d1 + d2)
                vw[pl.ds(0 * STEP * 4 + k * STEP + o, L)] = jnp.where(m, d0, 0.0)
                vw[pl.ds(1 * STEP * 4 + k * STEP + o, L)] = jnp.where(m, d1, 0.0)
                vw[pl.ds(2 * STEP * 4 + k * STEP + o, L)] = jnp.where(m, d2, 0.0)
            for j in range(3):
                vw[pl.ds(j * STEP * 4 + 3 * STEP + o, L)] = area_m
            return 0

        lax.fori_loop(0, GRP, math_body, 0)

    for b01 in range(2):
        soff = b01 * V

        build_gidx(0, 0, soff)
        fire_g(0)

        def pair_body(i, _, soff=soff):
            for sub in range(2):
                t = 2 * i + sub
                p = sub
                q = 1 - sub
                if sub == 0:
                    build_gidx(q, t + 1, soff)
                    fire_g(q)
                else:
                    @pl.when(i < NSTEPS // 2 - 1)
                    def _():
                        build_gidx(q, t + 1, soff)
                        fire_g(q)

                @pl.when(i > 0)
                def _():
                    wait_s(p)

                wait_g(p)
                do_math(p, t, soff)
                fire_s(p)
            return 0

        lax.fori_loop(0, NSTEPS // 2, pair_body, 0)
        wait_s(0)
        wait_s(1)

    plsc.subcore_barrier()

    for m in range(4):

        @pl.when(s < NS - 1)
        def _():
            pltpu.sync_copy(accum.at[pl.ds(m * 2 * V + s * VCH, VCH)],
                            abuf.at[pl.ds(m * VCH, VCH)])

        @pl.when(s == NS - 1)
        def _():
            pltpu.sync_copy(
                accum.at[pl.ds(m * 2 * V + (NS - 1) * VCH, VTAIL)],
                abuf.at[pl.ds(m * VCH, VTAIL)])

    nvalid = jnp.where(s == NS - 1, VTAIL, VCH)
    ngroups = VCH // L

    def vtx_body(g, acc):
        o = g * L
        rv = o + iota
        valid = rv < nvalid
        ax = abuf[pl.ds(0 * VCH + o, L)]
        ay = abuf[pl.ds(1 * VCH + o, L)]
        az = abuf[pl.ds(2 * VCH + o, L)]
        aa = abuf[pl.ds(3 * VCH + o, L)]
        w = jnp.where(aa > 0, 0.0625 / aa, 0.0)
        lx = ax * w
        ly = ay * w
        lz = az * w
        sq = lx * lx + ly * ly + lz * lz
        cv = _sqrt_pos(sq)
        cv = jnp.where(valid, cv, 0.0)
        return acc + cv

    csum = lax.fori_loop(0, ngroups, vtx_body, jnp.zeros((L,), jnp.float32))
    svmem[pl.ds(0, L)] = csum
    pltpu.sync_copy(svmem, out.at[pl.ds((c * NS + s) * L, L)])


_lap_kernel = functools.partial(
    pl.kernel,
    out_type=jax.ShapeDtypeStruct((NC * NS * L,), jnp.float32),
    mesh=plsc.VectorSubcoreMesh(
        core_axis_name="c", subcore_axis_name="s",
        num_cores=NC, num_subcores=NS),
    scratch_types=[
        pltpu.VMEM((3 * FPT,), jnp.int32),
        [[[pltpu.VMEM((STEP,), jnp.int32) for _ in range(3)]
          for _ in range(3)] for _ in range(2)],
        [[[pltpu.VMEM((STEP,), jnp.int32) for _ in range(4)]
          for _ in range(3)] for _ in range(2)],
        [pltpu.VMEM((9 * STEP,), jnp.float32)
         for _ in range(2)],
        [pltpu.VMEM((12 * STEP,), jnp.float32)
         for _ in range(2)],
        pltpu.VMEM((L,), jnp.float32),
        pltpu.VMEM((4 * VCH,), jnp.float32),
        pltpu.VMEM_SHARED((6 * V,), jnp.float32),
        pltpu.VMEM_SHARED((8 * V,), jnp.float32),
        pltpu.SemaphoreType.DMA,
        pltpu.SemaphoreType.DMA,
        pltpu.SemaphoreType.DMA,
        pltpu.SemaphoreType.DMA,
    ],
)(_lap_body)


def kernel(x, y, faces):
    del y
    xt = jnp.transpose(x, (0, 2, 1)).reshape(-1)
    ft = jnp.pad(faces.T, ((0, 0), (0, F_PAD - F))).reshape(-1)
    zz = jnp.zeros((APT,), jnp.float32)
    parts = _lap_kernel(xt, ft, zz)
    return jnp.sum(parts) / (B * V)


# speedup vs baseline: 1.6483x; 1.0074x over previous
import functools

import jax
import jax.numpy as jnp
from jax import lax
from jax.experimental import pallas as pl
from jax.experimental.pallas import tpu as pltpu
from jax.experimental.pallas import tpu_sc as plsc

B = 4
V = 50000
F = 100000
NC = 2
NS = 16
L = 16
FPT = 6336
F_PAD = NS * FPT
STEP = 48
GRP = STEP // L
NSTEPS = FPT // STEP
APT = 8 * V // NS
VCH = 6256
VTAIL = 2 * V - (NS - 1) * VCH
XCH = 3200


def _iota():
    return lax.iota(jnp.int32, L)


def _rsqrt(x):
    i = lax.bitcast_convert_type(x, jnp.int32)
    i = jnp.int32(0x5F3759DF) - lax.shift_right_logical(
        i, jnp.full((L,), 1, jnp.int32))
    r = lax.bitcast_convert_type(i, jnp.float32)
    xh = 0.5 * x
    for _ in range(3):
        r = r * (1.5 - xh * r * r)
    return r


def _sqrt_pos(x):
    r = _rsqrt(jnp.maximum(x, 1e-30))
    return jnp.where(x > 0, x * r, 0.0)


def _lap_body(xt, ft, zz, out, fidx, gidx, sidx, rows, vals, svmem, abuf,
              xsrc, accum, sg0, sg1, ss0, ss1):
    sg = (sg0, sg1)
    ss = (ss0, ss1)
    c = lax.axis_index("c")
    s = lax.axis_index("s")
    iota = _iota()

    pltpu.sync_copy(zz, abuf.at[pl.ds(0, APT)])
    pltpu.sync_copy(abuf.at[pl.ds(0, APT)], accum.at[pl.ds(s * APT, APT)])

    for b01 in range(2):
        for m in range(3):
            src_base = (2 * c + b01) * 3 * V + m * V
            dst_base = m * 2 * V + b01 * V

            @pl.when(s < NS - 1)
            def _():
                pltpu.sync_copy(xt.at[pl.ds(src_base + s * XCH, XCH)],
                                abuf.at[pl.ds(0, XCH)])
                pltpu.sync_copy(abuf.at[pl.ds(0, XCH)],
                                xsrc.at[pl.ds(dst_base + s * XCH, XCH)])

            @pl.when(s == NS - 1)
            def _():
                tail = V - (NS - 1) * XCH
                pltpu.sync_copy(xt.at[pl.ds(src_base + (NS - 1) * XCH, tail)],
                                abuf.at[pl.ds(0, tail)])
                pltpu.sync_copy(abuf.at[pl.ds(0, tail)],
                                xsrc.at[pl.ds(dst_base + (NS - 1) * XCH, tail)])

    for j in range(3):
        pltpu.sync_copy(ft.at[pl.ds(j * F_PAD + s * FPT, FPT)],
                        fidx.at[pl.ds(j * FPT, FPT)])

    plsc.subcore_barrier()

    def build_gidx(p, t, soff):
        base = t * STEP

        def idx_body(g, _):
            o = g * L
            for j in range(3):
                fv = fidx[pl.ds(j * FPT + base + o, L)] + soff
                for m in range(3):
                    gidx[p][j][m][pl.ds(o, L)] = fv + m * (2 * V)
            return 0

        lax.fori_loop(0, GRP, idx_body, 0)

    def fire_g(p):
        for j in range(3):
            for m in range(3):
                pltpu.async_copy(
                    xsrc.at[gidx[p][j][m]],
                    rows[p].at[pl.ds((j * 3 + m) * STEP, STEP)], sg[p])

    def wait_g(p):
        for j in range(3):
            for m in range(3):
                pltpu.make_async_copy(
                    xsrc.at[gidx[p][j][m]],
                    rows[p].at[pl.ds((j * 3 + m) * STEP, STEP)],
                    sg[p]).wait()

    def fire_s(p):
        for j in range(3):
            for m in range(4):
                pltpu.async_copy(
                    vals[p].at[pl.ds((j * 4 + m) * STEP, STEP)],
                    accum.at[sidx[p][j][m]], ss[p], add=True)

    def wait_s(p):
        for j in range(3):
            for m in range(4):
                pltpu.make_async_copy(
                    vals[p].at[pl.ds((j * 4 + m) * STEP, STEP)],
                    accum.at[sidx[p][j][m]], ss[p]).wait()

    def do_math(p, t, soff):
        base = t * STEP
        rw = rows[p]
        vw = vals[p]

        def math_body(g, _):
            o = g * L
            fvs = [fidx[pl.ds(j * FPT + base + o, L)] + soff for j in range(3)]
            for j in range(3):
                for m in range(4):
                    sidx[p][j][m][pl.ds(o, L)] = fvs[j] + m * (2 * V)

            def ld(j, m):
                return rw[pl.ds((j * 3 + m) * STEP + o, L)]

            x0 = [ld(0, m) for m in range(3)]
            x1 = [ld(1, m) for m in range(3)]
            x2 = [ld(2, m) for m in range(3)]
            eA = [x1[k] - x2[k] for k in range(3)]
            eB = [x0[k] - x2[k] for k in range(3)]
            eC = [x0[k] - x1[k] for k in range(3)]
            A2 = eA[0] * eA[0] + eA[1] * eA[1] + eA[2] * eA[2]
            B2 = eB[0] * eB[0] + eB[1] * eB[1] + eB[2] * eB[2]
            C2 = eC[0] * eC[0] + eC[1] * eC[1] + eC[2] * eC[2]
            An = _sqrt_pos(A2)
            Bn = _sqrt_pos(B2)
            Cn = _sqrt_pos(C2)
            sh = 0.5 * (An + Bn + Cn)
            prod = sh * (sh - An) * (sh - Bn) * (sh - Cn)
            prod = jnp.maximum(prod, 1e-12)
            rinv = _rsqrt(prod)
            area = prod * rinv
            cota = (B2 + C2 - A2) * rinv
            cotb = (A2 + C2 - B2) * rinv
            cotc = (A2 + B2 - C2) * rinv
            gid = s * FPT + base + o + iota
            m = gid < F
            area_m = jnp.where(m, area, 0.0)
            for k in range(3):
                d1 = cotc * (x0[k] - x1[k]) + cota * (x2[k] - x1[k])
                d2 = cota * (x1[k] - x2[k]) + cotb * (x0[k] - x2[k])
                d0 = -(d1 + d2)
                vw[pl.ds(0 * STEP * 4 + k * STEP + o, L)] = jnp.where(m, d0, 0.0)
                vw[pl.ds(1 * STEP * 4 + k * STEP + o, L)] = jnp.where(m, d1, 0.0)
                vw[pl.ds(2 * STEP * 4 + k * STEP + o, L)] = jnp.where(m, d2, 0.0)
            for j in range(3):
                vw[pl.ds(j * STEP * 4 + 3 * STEP + o, L)] = area_m
            return 0

        lax.fori_loop(0, GRP, math_body, 0)

    for b01 in range(2):
        soff = b01 * V

        build_gidx(0, 0, soff)
        fire_g(0)

        def pair_body(i, _, soff=soff):
            for sub in range(2):
                t = 2 * i + sub
                p = sub
                q = 1 - sub
                if sub == 0:
                    build_gidx(q, t + 1, soff)
                    fire_g(q)
                else:
                    @pl.when(i < NSTEPS // 2 - 1)
                    def _():
                        build_gidx(q, t + 1, soff)
                        fire_g(q)

                @pl.when(i > 0)
                def _():
                    wait_s(p)

                wait_g(p)
                do_math(p, t, soff)
                fire_s(p)
            return 0

        lax.fori_loop(0, NSTEPS // 2, pair_body, 0)
        wait_s(0)
        wait_s(1)

    plsc.subcore_barrier()

    for m in range(4):

        @pl.when(s < NS - 1)
        def _():
            pltpu.sync_copy(accum.at[pl.ds(m * 2 * V + s * VCH, VCH)],
                            abuf.at[pl.ds(m * VCH, VCH)])

        @pl.when(s == NS - 1)
        def _():
            pltpu.sync_copy(
                accum.at[pl.ds(m * 2 * V + (NS - 1) * VCH, VTAIL)],
                abuf.at[pl.ds(m * VCH, VTAIL)])

    nvalid = jnp.where(s == NS - 1, VTAIL, VCH)
    ngroups = VCH // L

    def vtx_body(g, acc):
        o = g * L
        rv = o + iota
        valid = rv < nvalid
        ax = abuf[pl.ds(0 * VCH + o, L)]
        ay = abuf[pl.ds(1 * VCH + o, L)]
        az = abuf[pl.ds(2 * VCH + o, L)]
        aa = abuf[pl.ds(3 * VCH + o, L)]
        w = jnp.where(aa > 0, 0.0625 / aa, 0.0)
        lx = ax * w
        ly = ay * w
        lz = az * w
        sq = lx * lx + ly * ly + lz * lz
        cv = _sqrt_pos(sq)
        cv = jnp.where(valid, cv, 0.0)
        return acc + cv

    csum = lax.fori_loop(0, ngroups, vtx_body, jnp.zeros((L,), jnp.float32))
    svmem[pl.ds(0, L)] = csum
    pltpu.sync_copy(svmem, out.at[pl.ds((c * NS + s) * L, L)])


_lap_kernel = functools.partial(
    pl.kernel,
    out_type=jax.ShapeDtypeStruct((NC * NS * L,), jnp.float32),
    mesh=plsc.VectorSubcoreMesh(
        core_axis_name="c", subcore_axis_name="s",
        num_cores=NC, num_subcores=NS),
    scratch_types=[
        pltpu.VMEM((3 * FPT,), jnp.int32),
        [[[pltpu.VMEM((STEP,), jnp.int32) for _ in range(3)]
          for _ in range(3)] for _ in range(2)],
        [[[pltpu.VMEM((STEP,), jnp.int32) for _ in range(4)]
          for _ in range(3)] for _ in range(2)],
        [pltpu.VMEM((9 * STEP,), jnp.float32)
         for _ in range(2)],
        [pltpu.VMEM((12 * STEP,), jnp.float32)
         for _ in range(2)],
        pltpu.VMEM((L,), jnp.float32),
        pltpu.VMEM((4 * VCH,), jnp.float32),
        pltpu.VMEM_SHARED((6 * V,), jnp.float32),
        pltpu.VMEM_SHARED((8 * V,), jnp.float32),
        pltpu.SemaphoreType.DMA,
        pltpu.SemaphoreType.DMA,
        pltpu.SemaphoreType.DMA,
        pltpu.SemaphoreType.DMA,
    ],
)(_lap_body)


def kernel(x, y, faces):
    del y
    xt = jnp.transpose(x, (0, 2, 1)).reshape(-1)
    ft = jnp.pad(faces.T, ((0, 0), (0, F_PAD - F))).reshape(-1)
    zz = jnp.zeros((APT,), jnp.float32)
    parts = _lap_kernel(xt, ft, zz)
    return jnp.sum(parts) / (B * V)
